# baseline (speedup 1.0000x reference)
import functools

import jax
import jax.numpy as jnp
from jax import lax
from jax.experimental import pallas as pl
from jax.experimental.pallas import tpu as pltpu
from jax.experimental.pallas import tpu_sc as plsc

_EPS = 1e-12
_CHUNK = 128
_NGRP = _CHUNK // 16
_NBUF = 2


def _rsqrt(x):
    i = lax.bitcast_convert_type(x, jnp.int32)
    i = jnp.int32(0x5F3759DF) - lax.shift_right_logical(i, 1)
    y = lax.bitcast_convert_type(i, jnp.float32)
    hx = 0.5 * x
    for _ in range(3):
        y = y * (1.5 - hx * y * y)
    return y


@functools.partial(jax.jit, static_argnums=(5, 6, 7))
def _emb(ids_flat, word_table, pos_table, ln_gamma, ln_beta, seq_len, nc, ns):
    BL = ids_flat.shape[0]
    H = word_table.shape[1]
    NW = nc * ns
    tokens_per_w = BL // NW
    n_chunks = tokens_per_w // _CHUNK
    KB = H // 16

    mesh = plsc.VectorSubcoreMesh(
        core_axis_name="c", subcore_axis_name="s", num_cores=nc, num_subcores=ns
    )

    @functools.partial(
        pl.kernel,
        out_type=jax.ShapeDtypeStruct((BL, H), jnp.float32),
        mesh=mesh,
        scratch_types=[
            pltpu.VMEM((2 * seq_len, H), jnp.float32),
            pltpu.VMEM((H,), jnp.float32),
            pltpu.VMEM((H,), jnp.float32),
            pltpu.VMEM((_NBUF, _CHUNK), jnp.int32),
            pltpu.VMEM((_NBUF, _CHUNK, H), jnp.float32),
            pltpu.VMEM((_NBUF, _CHUNK, H), jnp.float32),
            pltpu.SemaphoreType.DMA((_NBUF,)),
            pltpu.SemaphoreType.DMA((_NBUF,)),
            pltpu.SemaphoreType.DMA((_NBUF,)),
        ],
        compiler_params=pltpu.CompilerParams(
            use_tc_tiling_on_sc=False, needs_layout_passes=False),
    )
    def emb(ids_hbm, word_hbm, pos_hbm, g_hbm, b_hbm, out_hbm,
            pos_v, g_v, b_v, idx_v, rows_v, obuf_v, ids_sem, gat_sem, out_sem):
        wid = lax.axis_index("s") * nc + lax.axis_index("c")
        pltpu.sync_copy(pos_hbm.at[pl.ds(0, seq_len)], pos_v.at[pl.ds(0, seq_len)])
        pltpu.sync_copy(pos_hbm.at[pl.ds(0, seq_len)], pos_v.at[pl.ds(seq_len, seq_len)])
        pltpu.sync_copy(g_hbm, g_v)
        pltpu.sync_copy(b_hbm, b_v)
        lanes = lax.iota(jnp.int32, 16)
        ridx = [lanes + (g * 16) for g in range(_NGRP)]
        base0 = wid * tokens_per_w

        def ids_copy(c, b):
            return pltpu.make_async_copy(
                ids_hbm.at[pl.ds(base0 + c * _CHUNK, _CHUNK)], idx_v.at[b],
                ids_sem.at[b])

        def gather(b):
            return pltpu.make_async_copy(
                word_hbm.at[idx_v.at[b]], rows_v.at[b], gat_sem.at[b])

        def out_copy(c, b):
            return pltpu.make_async_copy(
                obuf_v.at[b], out_hbm.at[pl.ds(base0 + c * _CHUNK, _CHUNK)],
                out_sem.at[b])

        for b in range(_NBUF):
            ids_copy(b, b).start()
        for b in range(_NBUF):
            ids_copy(b, b).wait()
            gather(b).start()

        def outer_body(c2, _):
            for b in range(_NBUF):
                c = c2 * _NBUF + b
                gather(b).wait()
                @pl.when(c2 < (n_chunks // _NBUF) - 1)
                def _prefetch_ids():
                    ids_copy(c + _NBUF, b).start()

                @pl.when(c2 > 0)
                def _drain_out():
                    out_copy(c, b).wait()

                p0 = lax.rem(c * _CHUNK, seq_len)
                pidx = [p0 + ridx[g] for g in range(_NGRP)]
                rows_b = rows_v.at[b]
                obuf_b = obuf_v.at[b]

                def stats_body(f, acc):
                    ss, qq = acc
                    fv = jnp.full((16,), 0, jnp.int32) + f
                    ss2, qq2 = [], []
                    for g in range(_NGRP):
                        w = plsc.load_gather(rows_b, [ridx[g], fv])
                        p = plsc.load_gather(pos_v, [pidx[g], fv])
                        x = w + p
                        plsc.store_scatter(rows_b, [ridx[g], fv], x)
                        ss2.append(ss[g] + x)
                        qq2.append(qq[g] + x * x)
                    return (tuple(ss2), tuple(qq2))

                zero = jnp.zeros((16,), jnp.float32)
                ss, qq = lax.fori_loop(
                    0, H, stats_body,
                    (tuple(zero for _ in range(_NGRP)),
                     tuple(zero for _ in range(_NGRP))))
                means, rstds = [], []
                for g in range(_NGRP):
                    mean = ss[g] * (1.0 / H)
                    var = qq[g] * (1.0 / H) - mean * mean
                    means.append(mean)
                    rstds.append(_rsqrt(var + _EPS))

                def norm_body(f, _):
                    fv = jnp.full((16,), 0, jnp.int32) + f
                    gf = plsc.load_gather(g_v, [fv])
                    bf = plsc.load_gather(b_v, [fv])
                    for g in range(_NGRP):
                        x = plsc.load_gather(rows_b, [ridx[g], fv])
                        o = (x - means[g]) * rstds[g] * gf + bf
                        plsc.store_scatter(obuf_b, [ridx[g], fv], o)
                    return 0

                lax.fori_loop(0, H, norm_body, 0)
                out_copy(c, b).start()

                @pl.when(c2 < (n_chunks // _NBUF) - 1)
                def _prefetch_gather():
                    ids_copy(c + _NBUF, b).wait()
                    gather(b).start()
            return 0

        lax.fori_loop(0, n_chunks // _NBUF, outer_body, 0)
        for b in range(_NBUF):
            out_copy(n_chunks - _NBUF + b, b).wait()

    return emb(ids_flat, word_table, pos_table, ln_gamma, ln_beta)


def kernel(input_ids, word_table, pos_table, ln_gamma, ln_beta):
    B, L = input_ids.shape
    H = word_table.shape[1]
    try:
        info = plsc.get_sparse_core_info()
        nc, ns = info.num_cores, info.num_subcores
    except Exception:
        nc, ns = 2, 16
    ids_flat = input_ids.reshape(-1).astype(jnp.int32)
    out = _emb(ids_flat, word_table, pos_table, ln_gamma, ln_beta, L, nc, ns)
    return out.reshape(B, L, H)


# speedup vs baseline: 3.4004x; 3.4004x over previous
import functools

import jax
import jax.numpy as jnp
from jax import lax
from jax.experimental import pallas as pl
from jax.experimental.pallas import tpu as pltpu
from jax.experimental.pallas import tpu_sc as plsc

_EPS = 1e-12
_CHUNK = 128
_UNROLL = 4
_NBUF = 2


def _butterfly_sum(v, perms):
    for p in perms:
        v = v + jnp.take_along_axis(v, p, axis=0, mode="promise_in_bounds")
    return v


def _rsqrt(x):
    i = lax.bitcast_convert_type(x, jnp.int32)
    i = jnp.int32(0x5F3759DF) - lax.shift_right_logical(i, 1)
    y = lax.bitcast_convert_type(i, jnp.float32)
    hx = 0.5 * x
    for _ in range(2):
        y = y * (1.5 - hx * y * y)
    return y


@functools.partial(jax.jit, static_argnums=(5, 6, 7))
def _emb(ids_flat, word_pairs, pos_table, ln_gamma, ln_beta, seq_len, nc, ns):
    BL = ids_flat.shape[0]
    H = word_pairs.shape[1] // 2
    NW = nc * ns
    tokens_per_w = BL // NW
    n_chunks = tokens_per_w // _CHUNK
    KB = H // 16
    POSR = seq_len + _CHUNK

    mesh = plsc.VectorSubcoreMesh(
        core_axis_name="c", subcore_axis_name="s", num_cores=nc, num_subcores=ns
    )

    @functools.partial(
        pl.kernel,
        out_type=jax.ShapeDtypeStruct((BL, H), jnp.float32),
        mesh=mesh,
        scratch_types=[
            pltpu.VMEM((POSR, H), jnp.float32),
            pltpu.VMEM((H,), jnp.float32),
            pltpu.VMEM((H,), jnp.float32),
            pltpu.VMEM((_NBUF, _CHUNK), jnp.int32),
            pltpu.VMEM((_NBUF, _CHUNK), jnp.int32),
            pltpu.VMEM((_NBUF, _CHUNK, 2 * H), jnp.float32),
            pltpu.VMEM((_NBUF, _CHUNK, H), jnp.float32),
            pltpu.SMEM((_NBUF, _CHUNK), jnp.int32),
            pltpu.SemaphoreType.DMA((_NBUF,)),
            pltpu.SemaphoreType.DMA((_NBUF,)),
            pltpu.SemaphoreType.DMA((_NBUF,)),
        ],
        compiler_params=pltpu.CompilerParams(needs_layout_passes=False),
    )
    def emb(ids_hbm, word_hbm, pos_hbm, g_hbm, b_hbm, out_hbm,
            pos_v, g_v, b_v, idx_v, pidx_v, rows_v, obuf_v, off_s,
            ids_sem, gat_sem, out_sem):
        wid = lax.axis_index("s") * nc + lax.axis_index("c")
        pltpu.sync_copy(pos_hbm.at[pl.ds(0, seq_len)], pos_v.at[pl.ds(0, seq_len)])
        pltpu.sync_copy(pos_hbm.at[pl.ds(0, POSR - seq_len)],
                        pos_v.at[pl.ds(seq_len, POSR - seq_len)])
        pltpu.sync_copy(g_hbm, g_v)
        pltpu.sync_copy(b_hbm, b_v)
        gs = [g_v[pl.ds(k * 16, 16)] for k in range(KB)]
        bs = [b_v[pl.ds(k * 16, 16)] for k in range(KB)]
        lanes = lax.iota(jnp.int32, 16)
        perms = [lax.bitwise_xor(lanes, jnp.int32(m)) for m in (8, 4, 2, 1)]
        base0 = wid * tokens_per_w

        def ids_copy(c, b):
            return pltpu.make_async_copy(
                ids_hbm.at[pl.ds(base0 + c * _CHUNK, _CHUNK)], idx_v.at[b],
                ids_sem.at[b])

        def gather(b):
            return pltpu.make_async_copy(
                word_hbm.at[pidx_v.at[b]], rows_v.at[b], gat_sem.at[b])

        def convert_ids(b):
            for k in range(_CHUNK // 16):
                v = idx_v[b, pl.ds(k * 16, 16)]
                pidx_v[b, pl.ds(k * 16, 16)] = lax.shift_right_logical(v, 1)
                ov = lax.bitwise_and(v, 1) * H
                for i in range(16):
                    off_s[b, k * 16 + i] = ov[i]

        def out_copy(c, b):
            return pltpu.make_async_copy(
                obuf_v.at[b], out_hbm.at[pl.ds(base0 + c * _CHUNK, _CHUNK)],
                out_sem.at[b])

        for b in range(_NBUF):
            ids_copy(b, b).start()
        for b in range(_NBUF):
            ids_copy(b, b).wait()
            convert_ids(b)
            gather(b).start()

        def outer_body(c2, _):
            for b in range(_NBUF):
                c = c2 * _NBUF + b
                gather(b).wait()
                @pl.when(c2 < (n_chunks // _NBUF) - 1)
                def _prefetch_ids():
                    ids_copy(c + _NBUF, b).start()

                @pl.when(c2 > 0)
                def _drain_out():
                    out_copy(c, b).wait()

                p0 = lax.rem(c * _CHUNK, seq_len)

                def grp_body(g, _):
                    for u in range(_UNROLL):
                        j = g * _UNROLL + u
                        pr = p0 + j
                        par = off_s[b, j]
                        xs = []
                        for k in range(KB):
                            w = rows_v[b, j, pl.ds(par + k * 16, 16)]
                            p = pos_v[pr, pl.ds(k * 16, 16)]
                            xs.append(w + p)
                        sv = (xs[0] + xs[1]) + (xs[2] + xs[3])
                        qv = (xs[0] * xs[0] + xs[1] * xs[1]) + (xs[2] * xs[2] + xs[3] * xs[3])
                        mean = _butterfly_sum(sv, perms) * (1.0 / H)
                        var = _butterfly_sum(qv, perms) * (1.0 / H) - mean * mean
                        rstd = _rsqrt(var + _EPS)
                        for k in range(KB):
                            o = (xs[k] - mean) * rstd * gs[k] + bs[k]
                            obuf_v[b, j, pl.ds(k * 16, 16)] = o
                    return 0

                lax.fori_loop(0, _CHUNK // _UNROLL, grp_body, 0)
                out_copy(c, b).start()

                @pl.when(c2 < (n_chunks // _NBUF) - 1)
                def _prefetch_gather():
                    ids_copy(c + _NBUF, b).wait()
                    convert_ids(b)
                    gather(b).start()
            return 0

        lax.fori_loop(0, n_chunks // _NBUF, outer_body, 0)
        for b in range(_NBUF):
            out_copy(n_chunks - _NBUF + b, b).wait()

    return emb(ids_flat, word_pairs, pos_table, ln_gamma, ln_beta)


def kernel(input_ids, word_table, pos_table, ln_gamma, ln_beta):
    B, L = input_ids.shape
    V, H = word_table.shape
    try:
        info = plsc.get_sparse_core_info()
        nc, ns = info.num_cores, info.num_subcores
    except Exception:
        nc, ns = 2, 16
    ids_flat = input_ids.reshape(-1).astype(jnp.int32)
    word_pairs = word_table.reshape(V // 2, 2 * H)
    out = _emb(ids_flat, word_pairs, pos_table, ln_gamma, ln_beta, L, nc, ns)
    return out.reshape(B, L, H)


# speedup vs baseline: 3.4257x; 1.0074x over previous
import functools

import jax
import jax.numpy as jnp
from jax import lax
from jax.experimental import pallas as pl
from jax.experimental.pallas import tpu as pltpu
from jax.experimental.pallas import tpu_sc as plsc

_EPS = 1e-12
_CHUNK = 128
_UNROLL = 8
_NBUF = 2


def _butterfly_sum(v, perms):
    for p in perms:
        v = v + jnp.take_along_axis(v, p, axis=0, mode="promise_in_bounds")
    return v


def _rsqrt(x):
    i = lax.bitcast_convert_type(x, jnp.int32)
    i = jnp.int32(0x5F3759DF) - lax.shift_right_logical(i, 1)
    y = lax.bitcast_convert_type(i, jnp.float32)
    hx = 0.5 * x
    for _ in range(2):
        y = y * (1.5 - hx * y * y)
    return y


@functools.partial(jax.jit, static_argnums=(5, 6, 7))
def _emb(ids_flat, word_pairs, pos_table, ln_gamma, ln_beta, seq_len, nc, ns):
    BL = ids_flat.shape[0]
    H = word_pairs.shape[1] // 2
    NW = nc * ns
    tokens_per_w = BL // NW
    n_chunks = tokens_per_w // _CHUNK
    KB = H // 16
    POSR = seq_len + _CHUNK

    mesh = plsc.VectorSubcoreMesh(
        core_axis_name="c", subcore_axis_name="s", num_cores=nc, num_subcores=ns
    )

    @functools.partial(
        pl.kernel,
        out_type=jax.ShapeDtypeStruct((BL, H), jnp.float32),
        mesh=mesh,
        scratch_types=[
            pltpu.VMEM((POSR, H), jnp.float32),
            pltpu.VMEM((H,), jnp.float32),
            pltpu.VMEM((H,), jnp.float32),
            pltpu.VMEM((_NBUF, _CHUNK), jnp.int32),
            pltpu.VMEM((_NBUF, _CHUNK), jnp.int32),
            pltpu.VMEM((_NBUF, _CHUNK, 2 * H), jnp.float32),
            pltpu.VMEM((_NBUF, _CHUNK, H), jnp.float32),
            pltpu.SMEM((_NBUF, _CHUNK), jnp.int32),
            pltpu.SemaphoreType.DMA((_NBUF,)),
            pltpu.SemaphoreType.DMA((_NBUF,)),
            pltpu.SemaphoreType.DMA((_NBUF,)),
        ],
        compiler_params=pltpu.CompilerParams(needs_layout_passes=False),
    )
    def emb(ids_hbm, word_hbm, pos_hbm, g_hbm, b_hbm, out_hbm,
            pos_v, g_v, b_v, idx_v, pidx_v, rows_v, obuf_v, off_s,
            ids_sem, gat_sem, out_sem):
        wid = lax.axis_index("s") * nc + lax.axis_index("c")
        pltpu.sync_copy(pos_hbm.at[pl.ds(0, seq_len)], pos_v.at[pl.ds(0, seq_len)])
        pltpu.sync_copy(pos_hbm.at[pl.ds(0, POSR - seq_len)],
                        pos_v.at[pl.ds(seq_len, POSR - seq_len)])
        pltpu.sync_copy(g_hbm, g_v)
        pltpu.sync_copy(b_hbm, b_v)
        gs = [g_v[pl.ds(k * 16, 16)] for k in range(KB)]
        bs = [b_v[pl.ds(k * 16, 16)] for k in range(KB)]
        lanes = lax.iota(jnp.int32, 16)
        perms = [lax.bitwise_xor(lanes, jnp.int32(m)) for m in (8, 4, 2, 1)]
        base0 = wid * tokens_per_w

        def ids_copy(c, b):
            return pltpu.make_async_copy(
                ids_hbm.at[pl.ds(base0 + c * _CHUNK, _CHUNK)], idx_v.at[b],
                ids_sem.at[b])

        def gather(b):
            return pltpu.make_async_copy(
                word_hbm.at[pidx_v.at[b]], rows_v.at[b], gat_sem.at[b])

        def convert_ids(b):
            for k in range(_CHUNK // 16):
                v = idx_v[b, pl.ds(k * 16, 16)]
                pidx_v[b, pl.ds(k * 16, 16)] = lax.shift_right_logical(v, 1)
                ov = lax.bitwise_and(v, 1) * H
                for i in range(16):
                    off_s[b, k * 16 + i] = ov[i]

        def out_copy(c, b):
            return pltpu.make_async_copy(
                obuf_v.at[b], out_hbm.at[pl.ds(base0 + c * _CHUNK, _CHUNK)],
                out_sem.at[b])

        for b in range(_NBUF):
            ids_copy(b, b).start()
        for b in range(_NBUF):
            ids_copy(b, b).wait()
            convert_ids(b)
            gather(b).start()

        def outer_body(c2, _):
            for b in range(_NBUF):
                c = c2 * _NBUF + b
                gather(b).wait()
                @pl.when(c2 < (n_chunks // _NBUF) - 1)
                def _prefetch_ids():
                    ids_copy(c + _NBUF, b).start()

                @pl.when(c2 > 0)
                def _drain_out():
                    out_copy(c, b).wait()

                p0 = lax.rem(c * _CHUNK, seq_len)

                def grp_body(g, _):
                    for u in range(_UNROLL):
                        j = g * _UNROLL + u
                        pr = p0 + j
                        par = off_s[b, j]
                        xs = []
                        for k in range(KB):
                            w = rows_v[b, j, pl.ds(par + k * 16, 16)]
                            p = pos_v[pr, pl.ds(k * 16, 16)]
                            xs.append(w + p)
                        sv = (xs[0] + xs[1]) + (xs[2] + xs[3])
                        qv = (xs[0] * xs[0] + xs[1] * xs[1]) + (xs[2] * xs[2] + xs[3] * xs[3])
                        mean = _butterfly_sum(sv, perms) * (1.0 / H)
                        var = _butterfly_sum(qv, perms) * (1.0 / H) - mean * mean
                        rstd = _rsqrt(var + _EPS)
                        for k in range(KB):
                            o = (xs[k] - mean) * rstd * gs[k] + bs[k]
                            obuf_v[b, j, pl.ds(k * 16, 16)] = o
                    return 0

                lax.fori_loop(0, _CHUNK // _UNROLL, grp_body, 0)
                out_copy(c, b).start()

                @pl.when(c2 < (n_chunks // _NBUF) - 1)
                def _prefetch_gather():
                    ids_copy(c + _NBUF, b).wait()
                    convert_ids(b)
                    gather(b).start()
            return 0

        lax.fori_loop(0, n_chunks // _NBUF, outer_body, 0)
        for b in range(_NBUF):
            out_copy(n_chunks - _NBUF + b, b).wait()

    return emb(ids_flat, word_pairs, pos_table, ln_gamma, ln_beta)


def kernel(input_ids, word_table, pos_table, ln_gamma, ln_beta):
    B, L = input_ids.shape
    V, H = word_table.shape
    try:
        info = plsc.get_sparse_core_info()
        nc, ns = info.num_cores, info.num_subcores
    except Exception:
        nc, ns = 2, 16
    ids_flat = input_ids.reshape(-1).astype(jnp.int32)
    word_pairs = word_table.reshape(V // 2, 2 * H)
    out = _emb(ids_flat, word_pairs, pos_table, ln_gamma, ln_beta, L, nc, ns)
    return out.reshape(B, L, H)


# speedup vs baseline: 5.0761x; 1.4818x over previous
import functools

import jax
import jax.numpy as jnp
from jax import lax
from jax.experimental import pallas as pl
from jax.experimental.pallas import tpu as pltpu
from jax.experimental.pallas import tpu_sc as plsc

_EPS = 1e-12
_CHUNK = 128
_UNROLL = 4
_PUNROLL = 1
_NBUF = 2


def _butterfly_sum(v, perms):
    for p in perms:
        v = v + jnp.take_along_axis(v, p, axis=0, mode="promise_in_bounds")
    return v


def _rsqrt(x):
    i = lax.bitcast_convert_type(x, jnp.int32)
    i = jnp.int32(0x5F3759DF) - lax.shift_right_logical(i, 1)
    y = lax.bitcast_convert_type(i, jnp.float32)
    hx = 0.5 * x
    for _ in range(2):
        y = y * (1.5 - hx * y * y)
    return y


@functools.partial(jax.jit, static_argnums=(5, 6, 7))
def _emb(ids_flat, word_pairs, pos_table, ln_gamma, ln_beta, seq_len, nc, ns):
    BL = ids_flat.shape[0]
    H = word_pairs.shape[1] // 2
    NW = nc * ns
    tokens_per_w = BL // NW
    n_chunks = tokens_per_w // _CHUNK
    KB = H // 16
    POSR = seq_len + _CHUNK

    mesh = plsc.VectorSubcoreMesh(
        core_axis_name="c", subcore_axis_name="s", num_cores=nc, num_subcores=ns
    )

    @functools.partial(
        pl.kernel,
        out_type=jax.ShapeDtypeStruct((BL, H), jnp.float32),
        mesh=mesh,
        scratch_types=[
            pltpu.VMEM((POSR, H), jnp.float32),
            pltpu.VMEM((H,), jnp.float32),
            pltpu.VMEM((H,), jnp.float32),
            pltpu.VMEM((_NBUF, _CHUNK), jnp.int32),
            pltpu.VMEM((_NBUF, _CHUNK), jnp.int32),
            pltpu.VMEM((_NBUF, _CHUNK, 2 * H), jnp.float32),
            pltpu.VMEM((_NBUF, _CHUNK, H), jnp.float32),
            pltpu.SMEM((_NBUF, _CHUNK), jnp.int32),
            pltpu.SemaphoreType.DMA((_NBUF,)),
            pltpu.SemaphoreType.DMA((_NBUF,)),
            pltpu.SemaphoreType.DMA((_NBUF,)),
        ],
        compiler_params=pltpu.CompilerParams(needs_layout_passes=False),
    )
    def emb(ids_hbm, word_hbm, pos_hbm, g_hbm, b_hbm, out_hbm,
            pos_v, g_v, b_v, idx_v, pidx_v, rows_v, obuf_v, off_s,
            ids_sem, gat_sem, out_sem):
        wid = lax.axis_index("s") * nc + lax.axis_index("c")
        pltpu.sync_copy(pos_hbm.at[pl.ds(0, seq_len)], pos_v.at[pl.ds(0, seq_len)])
        pltpu.sync_copy(pos_hbm.at[pl.ds(0, POSR - seq_len)],
                        pos_v.at[pl.ds(seq_len, POSR - seq_len)])
        pltpu.sync_copy(g_hbm, g_v)
        pltpu.sync_copy(b_hbm, b_v)
        gs = [g_v[pl.ds(k * 16, 16)] for k in range(KB)]
        bs = [b_v[pl.ds(k * 16, 16)] for k in range(KB)]
        lanes = lax.iota(jnp.int32, 16)
        perms = [lax.bitwise_xor(lanes, jnp.int32(m)) for m in (8, 4, 2, 1)]
        base0 = wid * tokens_per_w

        def ids_copy(c, b):
            return pltpu.make_async_copy(
                ids_hbm.at[pl.ds(base0 + c * _CHUNK, _CHUNK)], idx_v.at[b],
                ids_sem.at[b])

        def gather(b):
            return pltpu.make_async_copy(
                word_hbm.at[pidx_v.at[b]], rows_v.at[b], gat_sem.at[b])

        def convert_ids(b):
            for k in range(_CHUNK // 16):
                v = idx_v[b, pl.ds(k * 16, 16)]
                pidx_v[b, pl.ds(k * 16, 16)] = lax.shift_right_logical(v, 1)
                ov = lax.bitwise_and(v, 1) * H
                for i in range(16):
                    off_s[b, k * 16 + i] = ov[i]

        def out_copy(c, b):
            return pltpu.make_async_copy(
                obuf_v.at[b], out_hbm.at[pl.ds(base0 + c * _CHUNK, _CHUNK)],
                out_sem.at[b])

        for b in range(_NBUF):
            ids_copy(b, b).start()
        for b in range(_NBUF):
            ids_copy(b, b).wait()
            convert_ids(b)
            gather(b).start()

        def outer_body(c2, _):
            for b in range(_NBUF):
                c = c2 * _NBUF + b
                gather(b).wait()
                @pl.when(c2 < (n_chunks // _NBUF) - 1)
                def _prefetch_ids():
                    ids_copy(c + _NBUF, b).start()

                @pl.when(c2 > 0)
                def _drain_out():
                    out_copy(c, b).wait()

                p0 = lax.rem(c * _CHUNK, seq_len)

                @plsc.parallel_loop(0, _CHUNK // _UNROLL, unroll=_PUNROLL)
                def grp_body(g):
                    for u in range(_UNROLL):
                        j = g * _UNROLL + u
                        pr = p0 + j
                        par = off_s[b, j]
                        xs = []
                        for k in range(KB):
                            w = rows_v[b, j, pl.ds(par + k * 16, 16)]
                            p = pos_v[pr, pl.ds(k * 16, 16)]
                            xs.append(w + p)
                        sv = (xs[0] + xs[1]) + (xs[2] + xs[3])
                        qv = (xs[0] * xs[0] + xs[1] * xs[1]) + (xs[2] * xs[2] + xs[3] * xs[3])
                        mean = _butterfly_sum(sv, perms) * (1.0 / H)
                        var = _butterfly_sum(qv, perms) * (1.0 / H) - mean * mean
                        rstd = _rsqrt(var + _EPS)
                        for k in range(KB):
                            o = (xs[k] - mean) * rstd * gs[k] + bs[k]
                            obuf_v[b, j, pl.ds(k * 16, 16)] = o

                out_copy(c, b).start()

                @pl.when(c2 < (n_chunks // _NBUF) - 1)
                def _prefetch_gather():
                    ids_copy(c + _NBUF, b).wait()
                    convert_ids(b)
                    gather(b).start()
            return 0

        lax.fori_loop(0, n_chunks // _NBUF, outer_body, 0)
        for b in range(_NBUF):
            out_copy(n_chunks - _NBUF + b, b).wait()

    return emb(ids_flat, word_pairs, pos_table, ln_gamma, ln_beta)


def kernel(input_ids, word_table, pos_table, ln_gamma, ln_beta):
    B, L = input_ids.shape
    V, H = word_table.shape
    try:
        info = plsc.get_sparse_core_info()
        nc, ns = info.num_cores, info.num_subcores
    except Exception:
        nc, ns = 2, 16
    ids_flat = input_ids.reshape(-1).astype(jnp.int32)
    word_pairs = word_table.reshape(V // 2, 2 * H)
    out = _emb(ids_flat, word_pairs, pos_table, ln_gamma, ln_beta, L, nc, ns)
    return out.reshape(B, L, H)
